# baseline (speedup 1.0000x reference)
import jax
import jax.numpy as jnp
from jax import lax
from jax.experimental import pallas as pl

E = 8
TOPK = 2
D = 768
DFF = 3072
R = 4
N = 4096

NB = 256
KPAD = 128

_DN_T = (((1,), (1,)), ((), ()))


def _moe_kernel(x_ref, gwt_ref, gb_ref, wi_ref, bi_ref, wo_ref, bo_ref,
                aT_ref, bflat_ref, out_ref):
    f32 = jnp.float32
    bf16 = jnp.bfloat16
    x = x_ref[...]

    logits = jnp.dot(x, gwt_ref[...], preferred_element_type=f32) + gb_ref[...]
    col = lax.broadcasted_iota(jnp.int32, (NB, KPAD), 1)
    valid = col < E
    logits = jnp.where(valid, logits, -1e30)
    m = jnp.max(logits, axis=-1, keepdims=True)
    p = jnp.where(valid, jnp.exp(logits - m), 0.0)
    scores = p / jnp.sum(p, axis=-1, keepdims=True)
    v1 = jnp.max(scores, axis=-1, keepdims=True)
    e1 = jnp.min(jnp.where(scores >= v1, col, KPAD), axis=-1, keepdims=True)
    scores2 = jnp.where(col == e1, -1.0, scores)
    v2 = jnp.max(scores2, axis=-1, keepdims=True)
    e2 = jnp.min(jnp.where(scores2 >= v2, col, KPAD), axis=-1, keepdims=True)
    w = v1 + v2

    xb = x.astype(bf16)
    U = jnp.dot(xb, aT_ref[...], preferred_element_type=f32)
    ecol = col // R
    S1 = jnp.where(ecol == e1, U, 0.0).astype(bf16)
    S2 = jnp.where(ecol == e2, U, 0.0).astype(bf16)
    S = jnp.concatenate([S1, S2], axis=0)
    L = jnp.dot(S, bflat_ref[...], preferred_element_type=f32)

    base = lax.dot_general(xb, wi_ref[...], _DN_T,
                           preferred_element_type=f32) + bi_ref[...]
    hsum = (jnp.maximum(base + L[:NB], 0) +
            jnp.maximum(base + L[NB:], 0)).astype(bf16)
    dn = lax.dot_general(hsum, wo_ref[...], _DN_T, preferred_element_type=f32)
    out_ref[...] = w * (dn + 2.0 * bo_ref[...])


def kernel(hidden_states, gate_W, gate_b, Wi, bi, Wo, bo, A, Bm):
    f32 = jnp.float32
    bf16 = jnp.bfloat16
    x = hidden_states.astype(f32)
    gwt = jnp.pad(gate_W.T, ((0, 0), (0, KPAD - E)))
    gb = jnp.pad(gate_b, (0, KPAD - E))[None, :]
    aT = jnp.pad(A.reshape(E * R, D).T,
                 ((0, 0), (0, KPAD - E * R))).astype(bf16)
    bflat = jnp.pad(jnp.transpose(Bm, (0, 2, 1)).reshape(E * R, DFF),
                    ((0, KPAD - E * R), (0, 0))).astype(bf16)

    out = pl.pallas_call(
        _moe_kernel,
        grid=(N // NB,),
        in_specs=[
            pl.BlockSpec((NB, D), lambda i: (i, 0)),
            pl.BlockSpec((D, KPAD), lambda i: (0, 0)),
            pl.BlockSpec((1, KPAD), lambda i: (0, 0)),
            pl.BlockSpec((DFF, D), lambda i: (0, 0)),
            pl.BlockSpec((1, DFF), lambda i: (0, 0)),
            pl.BlockSpec((D, DFF), lambda i: (0, 0)),
            pl.BlockSpec((1, D), lambda i: (0, 0)),
            pl.BlockSpec((D, KPAD), lambda i: (0, 0)),
            pl.BlockSpec((KPAD, DFF), lambda i: (0, 0)),
        ],
        out_specs=pl.BlockSpec((NB, D), lambda i: (i, 0)),
        out_shape=jax.ShapeDtypeStruct((N, D), f32),
    )(x, gwt, gb, Wi.astype(bf16), bi[None, :].astype(bf16), Wo.astype(bf16),
      bo[None, :], aT, bflat)
    return out


# speedup vs baseline: 1.1978x; 1.1978x over previous
import jax
import jax.numpy as jnp
from jax import lax
from jax.experimental import pallas as pl
from jax.experimental.pallas import tpu as pltpu

E = 8
TOPK = 2
D = 768
DFF = 3072
R = 4
N = 4096

NB = 512
KPAD = 128

_DN_T = (((1,), (1,)), ((), ()))


def _moe_kernel(x_ref, gwt_ref, gb_ref, wi_ref, bi_ref, wo_ref, bo_ref,
                aT_ref, bflat_ref, out_ref, wib_ref, wob_ref):
    f32 = jnp.float32
    bf16 = jnp.bfloat16

    @pl.when(pl.program_id(0) == 0)
    def _cast_weights():
        wib_ref[...] = wi_ref[...].astype(bf16)
        wob_ref[...] = wo_ref[...].astype(bf16)

    x = x_ref[...]

    logits = jnp.dot(x, gwt_ref[...], preferred_element_type=f32) + gb_ref[...]
    col = lax.broadcasted_iota(jnp.int32, (NB, KPAD), 1)
    valid = col < E
    logits = jnp.where(valid, logits, -1e30)
    m = jnp.max(logits, axis=-1, keepdims=True)
    p = jnp.where(valid, jnp.exp(logits - m), 0.0)
    scores = p / jnp.sum(p, axis=-1, keepdims=True)
    v1 = jnp.max(scores, axis=-1, keepdims=True)
    e1 = jnp.min(jnp.where(scores >= v1, col, KPAD), axis=-1, keepdims=True)
    scores2 = jnp.where(col == e1, -1.0, scores)
    v2 = jnp.max(scores2, axis=-1, keepdims=True)
    e2 = jnp.min(jnp.where(scores2 >= v2, col, KPAD), axis=-1, keepdims=True)
    w = v1 + v2

    xb = x.astype(bf16)
    U = jnp.dot(xb, aT_ref[...], preferred_element_type=f32)
    ecol = col // R
    S1 = jnp.where(ecol == e1, U, 0.0).astype(bf16)
    S2 = jnp.where(ecol == e2, U, 0.0).astype(bf16)
    S = jnp.concatenate([S1, S2], axis=0)
    L = jnp.dot(S, bflat_ref[...], preferred_element_type=f32)

    base = lax.dot_general(xb, wib_ref[...], _DN_T,
                           preferred_element_type=f32) + bi_ref[...]
    hsum = (jnp.maximum(base + L[:NB], 0) +
            jnp.maximum(base + L[NB:], 0)).astype(bf16)
    dn = lax.dot_general(hsum, wob_ref[...], _DN_T, preferred_element_type=f32)
    out_ref[...] = w * (dn + 2.0 * bo_ref[...])


def kernel(hidden_states, gate_W, gate_b, Wi, bi, Wo, bo, A, Bm):
    f32 = jnp.float32
    bf16 = jnp.bfloat16
    x = hidden_states.astype(f32)
    gwt = jnp.pad(gate_W.T, ((0, 0), (0, KPAD - E)))
    gb = jnp.pad(gate_b, (0, KPAD - E))[None, :]
    aT = jnp.pad(A.reshape(E * R, D).T,
                 ((0, 0), (0, KPAD - E * R))).astype(bf16)
    bflat = jnp.pad(jnp.transpose(Bm, (0, 2, 1)).reshape(E * R, DFF),
                    ((0, KPAD - E * R), (0, 0))).astype(bf16)

    out = pl.pallas_call(
        _moe_kernel,
        grid=(N // NB,),
        in_specs=[
            pl.BlockSpec((NB, D), lambda i: (i, 0)),
            pl.BlockSpec((D, KPAD), lambda i: (0, 0)),
            pl.BlockSpec((1, KPAD), lambda i: (0, 0)),
            pl.BlockSpec((DFF, D), lambda i: (0, 0)),
            pl.BlockSpec((1, DFF), lambda i: (0, 0)),
            pl.BlockSpec((D, DFF), lambda i: (0, 0)),
            pl.BlockSpec((1, D), lambda i: (0, 0)),
            pl.BlockSpec((D, KPAD), lambda i: (0, 0)),
            pl.BlockSpec((KPAD, DFF), lambda i: (0, 0)),
        ],
        out_specs=pl.BlockSpec((NB, D), lambda i: (i, 0)),
        out_shape=jax.ShapeDtypeStruct((N, D), f32),
        scratch_shapes=[
            pltpu.VMEM((DFF, D), bf16),
            pltpu.VMEM((D, DFF), bf16),
        ],
    )(x, gwt, gb, Wi, bi[None, :].astype(bf16), Wo, bo[None, :], aT, bflat)
    return out
